# baseline (speedup 1.0000x reference)
import functools

import jax
import jax.numpy as jnp
from jax import lax
from jax.experimental import pallas as pl
from jax.experimental.pallas import tpu as pltpu
from jax.experimental.pallas import tpu_sc as plsc

VOCAB = 49408
EMBED = 768
CTX = 77
BATCH = 1024
NTOK = BATCH * CTX

NC = 2
NS = 16
NW = NC * NS
PER_W = NTOK // NW

K = 8
NCHUNK = PER_W // K
NBUF = 4
LANES = 16
VECS = EMBED // LANES


def _sc_embed_body(x_hbm, tok_hbm, pos_hbm, out_hbm,
                   idx_v, pos_v, buf0, buf1, buf2, buf3,
                   g0, g1, g2, g3, o0, o1, o2, o3):
    bufs = (buf0, buf1, buf2, buf3)
    gsems = (g0, g1, g2, g3)
    osems = (o0, o1, o2, o3)

    wid = lax.axis_index("s") * NC + lax.axis_index("c")
    base = wid * PER_W

    pltpu.sync_copy(x_hbm.at[wid], idx_v)
    pltpu.sync_copy(pos_hbm, pos_v)

    def start_gather(c, b):
        return pltpu.async_copy(tok_hbm.at[idx_v.at[c]], bufs[b], gsems[b])

    def wait_gather(c, b):
        pltpu.make_async_copy(tok_hbm.at[idx_v.at[c]], bufs[b], gsems[b]).wait()

    def start_scatter(c, b):
        return pltpu.async_copy(
            bufs[b], out_hbm.at[pl.ds(base + c * K, K)], osems[b])

    def wait_scatter(c, b):
        pltpu.make_async_copy(
            bufs[b], out_hbm.at[pl.ds(base + c * K, K)], osems[b]).wait()

    def add_pos(c, b):
        buf = bufs[b]
        prow0 = c * K

        def row_body(r, carry):
            pr = lax.rem(prow0 + r, CTX)
            for i in range(VECS):
                sl = pl.ds(i * LANES, LANES)
                buf[r, sl] = buf[r, sl] + pos_v[pr, sl]
            return carry

        lax.fori_loop(0, K, row_body, 0, unroll=False)

    start_gather(0, 0)
    start_gather(1, 1)

    def step(c, b):
        @pl.when(c + 2 < NCHUNK)
        def _():
            nb = (b + 2) % NBUF

            @pl.when(c >= 2)
            def _():
                wait_scatter(c - 2, nb)

            start_gather(c + 2, nb)

        wait_gather(c, b)
        add_pos(c, b)
        start_scatter(c, b)

    def outer(i, carry):
        c0 = i * NBUF
        for bb in range(NBUF):
            step(c0 + bb, bb)
        return carry

    lax.fori_loop(0, NCHUNK // NBUF, outer, 0, unroll=False)

    for j in range(NBUF):
        c = NCHUNK - NBUF + j
        wait_scatter(c, c % NBUF)


@functools.partial(jax.jit, static_argnums=())
def _sc_embed(x3, token_embed, pos_embed):
    mesh = plsc.VectorSubcoreMesh(
        core_axis_name="c", subcore_axis_name="s",
        num_cores=NC, num_subcores=NS)
    fn = pl.kernel(
        _sc_embed_body,
        out_type=jax.ShapeDtypeStruct((NTOK, EMBED), jnp.float32),
        mesh=mesh,
        scratch_types=[
            pltpu.VMEM((NCHUNK, K), jnp.int32),
            pltpu.VMEM((CTX, EMBED), jnp.float32),
            pltpu.VMEM((K, EMBED), jnp.float32),
            pltpu.VMEM((K, EMBED), jnp.float32),
            pltpu.VMEM((K, EMBED), jnp.float32),
            pltpu.VMEM((K, EMBED), jnp.float32),
            pltpu.SemaphoreType.DMA,
            pltpu.SemaphoreType.DMA,
            pltpu.SemaphoreType.DMA,
            pltpu.SemaphoreType.DMA,
            pltpu.SemaphoreType.DMA,
            pltpu.SemaphoreType.DMA,
            pltpu.SemaphoreType.DMA,
            pltpu.SemaphoreType.DMA,
        ],
    )
    return fn(x3, token_embed, pos_embed)


def kernel(x, token_embed, pos_embed):
    x3 = x.astype(jnp.int32).reshape(NW, NCHUNK, K)
    out = _sc_embed(x3, token_embed, pos_embed)
    return out.reshape(BATCH, CTX, EMBED)


# speedup vs baseline: 1.4479x; 1.4479x over previous
import jax
import jax.numpy as jnp
from jax import lax
from jax.experimental import pallas as pl
from jax.experimental.pallas import tpu as pltpu
from jax.experimental.pallas import tpu_sc as plsc

VOCAB = 49408
EMBED = 768
CTX = 77
BATCH = 1024

NC = 2
NS = 16
NW = NC * NS
B_PER_W = BATCH // NW

CTXP = 80
K = 16
RO = (0, 16, 32, 48, 64)
ADDROWS = (16, 16, 16, 16, 13)
NCH = 5
NBUF = 5
TAILK = 8
LANES = 16
VECS = EMBED // LANES


def _sc_embed_body(x_hbm, tok_hbm, pos_hbm, out_hbm, tail_hbm,
                   idx_v, pos_v, buf0, buf1, buf2, buf3, buf4,
                   g0, g1, g2, g3, g4, o0, o1, o2, o3, o4):
    bufs = (buf0, buf1, buf2, buf3, buf4)
    gsems = (g0, g1, g2, g3, g4)
    osems = (o0, o1, o2, o3, o4)

    wid = lax.axis_index("s") * NC + lax.axis_index("c")
    b0 = wid * B_PER_W

    pltpu.sync_copy(x_hbm.at[pl.ds(b0, B_PER_W)], idx_v)
    pltpu.sync_copy(pos_hbm, pos_v)

    def start_gather(bb, j):
        pltpu.async_copy(
            tok_hbm.at[idx_v.at[bb, pl.ds(RO[j], K)]],
            bufs[j], gsems[j])

    def wait_gather(bb, j):
        pltpu.make_async_copy(
            tok_hbm.at[idx_v.at[bb, pl.ds(RO[j], K)]],
            bufs[j], gsems[j]).wait()

    def start_scatter(bb, j):
        if j < NCH - 1:
            pltpu.async_copy(
                bufs[j], out_hbm.at[b0 + bb, pl.ds(RO[j], K)], osems[j])
        else:
            pltpu.async_copy(
                bufs[j].at[pl.ds(0, TAILK)],
                out_hbm.at[b0 + bb, pl.ds(RO[j], TAILK)], osems[j])
            pltpu.async_copy(
                bufs[j].at[pl.ds(TAILK, TAILK)],
                tail_hbm.at[b0 + bb], osems[j])

    def wait_scatter(bb, j):
        if j < NCH - 1:
            pltpu.make_async_copy(
                bufs[j], out_hbm.at[b0 + bb, pl.ds(RO[j], K)],
                osems[j]).wait()
        else:
            pltpu.make_async_copy(
                bufs[j].at[pl.ds(0, TAILK)],
                out_hbm.at[b0 + bb, pl.ds(RO[j], TAILK)], osems[j]).wait()
            pltpu.make_async_copy(
                bufs[j].at[pl.ds(TAILK, TAILK)],
                tail_hbm.at[b0 + bb], osems[j]).wait()

    def add_pos(j):
        buf = bufs[j]
        r0 = RO[j]

        def row_body(r, carry):
            for i in range(VECS):
                sl = pl.ds(i * LANES, LANES)
                buf[r, sl] = buf[r, sl] + pos_v[r0 + r, sl]
            return carry

        lax.fori_loop(0, ADDROWS[j], row_body, 0, unroll=False)

    start_gather(0, 0)
    start_gather(0, 1)

    def chunk_step(bb, j):
        jl = (j + 2) % NCH
        dl = (j + 2) // NCH
        jw = (j - 3) % NCH
        dw = (j - 3) // NCH

        def lookahead():
            @pl.when(bb * NCH + j >= 3)
            def _():
                wait_scatter(bb + dw, jw)
            start_gather(bb + dl, jl)

        if dl == 0:
            lookahead()
        else:
            @pl.when(bb < B_PER_W - 1)
            def _():
                lookahead()

        wait_gather(bb, j)
        add_pos(j)
        start_scatter(bb, j)

    def seq_body(bb, carry):
        for j in range(NCH):
            chunk_step(bb, j)
        return carry

    lax.fori_loop(0, B_PER_W, seq_body, 0, unroll=False)

    for j in range(NCH):
        wait_scatter(B_PER_W - 1, j)


@jax.jit
def _sc_embed(x, token_embed, pos_embed):
    mesh = plsc.VectorSubcoreMesh(
        core_axis_name="c", subcore_axis_name="s",
        num_cores=NC, num_subcores=NS)
    fn = pl.kernel(
        _sc_embed_body,
        out_type=[
            jax.ShapeDtypeStruct((BATCH, CTX, EMBED), jnp.float32),
            jax.ShapeDtypeStruct((BATCH, TAILK, EMBED), jnp.float32),
        ],
        mesh=mesh,
        scratch_types=[
            pltpu.VMEM((B_PER_W, CTXP), jnp.int32),
            pltpu.VMEM((CTX, EMBED), jnp.float32),
            pltpu.VMEM((K, EMBED), jnp.float32),
            pltpu.VMEM((K, EMBED), jnp.float32),
            pltpu.VMEM((K, EMBED), jnp.float32),
            pltpu.VMEM((K, EMBED), jnp.float32),
            pltpu.VMEM((K, EMBED), jnp.float32),
            pltpu.SemaphoreType.DMA,
            pltpu.SemaphoreType.DMA,
            pltpu.SemaphoreType.DMA,
            pltpu.SemaphoreType.DMA,
            pltpu.SemaphoreType.DMA,
            pltpu.SemaphoreType.DMA,
            pltpu.SemaphoreType.DMA,
            pltpu.SemaphoreType.DMA,
            pltpu.SemaphoreType.DMA,
            pltpu.SemaphoreType.DMA,
        ],
    )
    return fn(x, token_embed, pos_embed)


def kernel(x, token_embed, pos_embed):
    xp = jnp.pad(x.astype(jnp.int32), ((0, 0), (0, CTXP - CTX)))
    out_main, out_tail = _sc_embed(xp, token_embed, pos_embed)
    ntail = CTX - (RO[-1] + TAILK)
    return lax.dynamic_update_slice(
        out_main, out_tail[:, :ntail, :], (0, RO[-1] + TAILK, 0))


# speedup vs baseline: 1.4826x; 1.0240x over previous
import jax
import jax.numpy as jnp
from jax import lax
from jax.experimental import pallas as pl
from jax.experimental.pallas import tpu as pltpu
from jax.experimental.pallas import tpu_sc as plsc

VOCAB = 49408
EMBED = 768
CTX = 77
BATCH = 1024

NC = 2
NS = 16
NW = NC * NS
B_PER_W = BATCH // NW

CTXP = 80
K = 16
RO = (0, 16, 32, 48, 64)
ADDROWS = (16, 16, 16, 16, 13)
NCH = 5
NBUF = 5
TAILK = 8
LANES = 16
VECS = EMBED // LANES


def _sc_embed_body(x_hbm, tok_hbm, pos_hbm, out_hbm, tail_hbm,
                   idx_v, pos_v, buf0, buf1, buf2, buf3, buf4,
                   g0, g1, g2, g3, g4, o0, o1, o2, o3, o4):
    bufs = (buf0, buf1, buf2, buf3, buf4)
    gsems = (g0, g1, g2, g3, g4)
    osems = (o0, o1, o2, o3, o4)

    wid = lax.axis_index("s") * NC + lax.axis_index("c")
    b0 = wid * B_PER_W

    pltpu.sync_copy(x_hbm.at[pl.ds(b0, B_PER_W)], idx_v)
    pltpu.sync_copy(pos_hbm, pos_v)

    def start_gather(bb, j):
        pltpu.async_copy(
            tok_hbm.at[idx_v.at[bb, pl.ds(RO[j], K)]],
            bufs[j], gsems[j])

    def wait_gather(bb, j):
        pltpu.make_async_copy(
            tok_hbm.at[idx_v.at[bb, pl.ds(RO[j], K)]],
            bufs[j], gsems[j]).wait()

    def start_scatter(bb, j):
        if j < NCH - 1:
            pltpu.async_copy(
                bufs[j], out_hbm.at[b0 + bb, pl.ds(RO[j], K)], osems[j])
        else:
            pltpu.async_copy(
                bufs[j].at[pl.ds(0, TAILK)],
                out_hbm.at[b0 + bb, pl.ds(RO[j], TAILK)], osems[j])
            pltpu.async_copy(
                bufs[j].at[pl.ds(TAILK, TAILK)],
                tail_hbm.at[b0 + bb], osems[j])

    def wait_scatter(bb, j):
        if j < NCH - 1:
            pltpu.make_async_copy(
                bufs[j], out_hbm.at[b0 + bb, pl.ds(RO[j], K)],
                osems[j]).wait()
        else:
            pltpu.make_async_copy(
                bufs[j].at[pl.ds(0, TAILK)],
                out_hbm.at[b0 + bb, pl.ds(RO[j], TAILK)], osems[j]).wait()
            pltpu.make_async_copy(
                bufs[j].at[pl.ds(TAILK, TAILK)],
                tail_hbm.at[b0 + bb], osems[j]).wait()

    def add_pos(j):
        buf = bufs[j]
        r0 = RO[j]

        def row_body(r, carry):
            for i in range(VECS):
                sl = pl.ds(i * LANES, LANES)
                buf[r, sl] = buf[r, sl] + pos_v[r0 + r, sl]
            return carry

        lax.fori_loop(0, ADDROWS[j], row_body, 0, unroll=False)

    start_gather(0, 0)
    start_gather(0, 1)

    def chunk_step(bb, j):
        jl = (j + 2) % NCH
        dl = (j + 2) // NCH
        jw = (j - 3) % NCH
        dw = (j - 3) // NCH

        def lookahead():
            @pl.when(bb * NCH + j >= 3)
            def _():
                wait_scatter(bb + dw, jw)
            start_gather(bb + dl, jl)

        if dl == 0:
            lookahead()
        else:
            @pl.when(bb < B_PER_W - 1)
            def _():
                lookahead()

        wait_gather(bb, j)
        add_pos(j)
        start_scatter(bb, j)

    def seq_body(bb, carry):
        for j in range(NCH):
            chunk_step(bb, j)
        return carry

    lax.fori_loop(0, B_PER_W, seq_body, 0, unroll=False)

    for j in range(NCH):
        wait_scatter(B_PER_W - 1, j)


@jax.jit
def _sc_embed(x, token_embed, pos_embed):
    mesh = plsc.VectorSubcoreMesh(
        core_axis_name="c", subcore_axis_name="s",
        num_cores=NC, num_subcores=NS)
    fn = pl.kernel(
        _sc_embed_body,
        out_type=[
            jax.ShapeDtypeStruct((BATCH, CTX, EMBED), jnp.float32),
            jax.ShapeDtypeStruct((BATCH, TAILK, EMBED), jnp.float32),
        ],
        mesh=mesh,
        scratch_types=[
            pltpu.VMEM((B_PER_W, CTXP), jnp.int32),
            pltpu.VMEM((CTX, EMBED), jnp.float32),
            pltpu.VMEM((K, EMBED), jnp.float32),
            pltpu.VMEM((K, EMBED), jnp.float32),
            pltpu.VMEM((K, EMBED), jnp.float32),
            pltpu.VMEM((K, EMBED), jnp.float32),
            pltpu.VMEM((K, EMBED), jnp.float32),
            pltpu.SemaphoreType.DMA,
            pltpu.SemaphoreType.DMA,
            pltpu.SemaphoreType.DMA,
            pltpu.SemaphoreType.DMA,
            pltpu.SemaphoreType.DMA,
            pltpu.SemaphoreType.DMA,
            pltpu.SemaphoreType.DMA,
            pltpu.SemaphoreType.DMA,
            pltpu.SemaphoreType.DMA,
            pltpu.SemaphoreType.DMA,
        ],
    )
    return fn(x, token_embed, pos_embed)


NTAIL = CTX - (RO[-1] + TAILK)
MB = 128


def _merge_body(main_ref, tail_ref, out_ref):
    del main_ref
    out_ref[...] = tail_ref[...]


def _merge_tail(out_main, out_tail):
    return pl.pallas_call(
        _merge_body,
        grid=(BATCH // MB,),
        in_specs=[
            pl.BlockSpec(memory_space=pl.ANY),
            pl.BlockSpec((MB, TAILK, EMBED), lambda i: (i, 0, 0)),
        ],
        out_specs=pl.BlockSpec(
            (MB, TAILK, EMBED), lambda i: (i, (RO[-1] + TAILK) // TAILK, 0)),
        out_shape=jax.ShapeDtypeStruct((BATCH, CTX, EMBED), jnp.float32),
        input_output_aliases={0: 0},
    )(out_main, out_tail)


def kernel(x, token_embed, pos_embed):
    xp = jnp.pad(x.astype(jnp.int32), ((0, 0), (0, CTXP - CTX)))
    out_main, out_tail = _sc_embed(xp, token_embed, pos_embed)
    return _merge_tail(out_main, out_tail)


# speedup vs baseline: 1.5929x; 1.0744x over previous
import jax
import jax.numpy as jnp
from jax import lax
from jax.experimental import pallas as pl
from jax.experimental.pallas import tpu as pltpu
from jax.experimental.pallas import tpu_sc as plsc

VOCAB = 49408
EMBED = 768
CTX = 77
BATCH = 1024

NC = 2
NS = 16
NW = NC * NS
B_PER_W = BATCH // NW

K = 16
HALVES = B_PER_W // K
NCHT = CTX * HALVES
NBUF = 4
MAIN = (NCHT // NBUF) * NBUF
LANES = 16
VECS = EMBED // LANES


def _sc_embed_body(x_hbm, tok_hbm, pos_hbm, out_hbm,
                   idx_v, pos_v, buf0, buf1, buf2, buf3,
                   g0, g1, g2, g3, o0, o1, o2, o3):
    bufs = (buf0, buf1, buf2, buf3)
    gsems = (g0, g1, g2, g3)
    osems = (o0, o1, o2, o3)

    wid = lax.axis_index("s") * NC + lax.axis_index("c")
    b0 = wid * B_PER_W

    pltpu.sync_copy(x_hbm.at[wid], idx_v)
    pltpu.sync_copy(pos_hbm, pos_v)

    def start_gather(c, b):
        t = c // HALVES
        h = c % HALVES
        pltpu.async_copy(
            tok_hbm.at[idx_v.at[t, pl.ds(h * K, K)]], bufs[b], gsems[b])

    def wait_gather(c, b):
        t = c // HALVES
        h = c % HALVES
        pltpu.make_async_copy(
            tok_hbm.at[idx_v.at[t, pl.ds(h * K, K)]], bufs[b],
            gsems[b]).wait()

    def start_scatter(c, b):
        t = c // HALVES
        h = c % HALVES
        pltpu.async_copy(
            bufs[b], out_hbm.at[t, pl.ds(b0 + h * K, K)], osems[b])

    def wait_scatter(c, b):
        t = c // HALVES
        h = c % HALVES
        pltpu.make_async_copy(
            bufs[b], out_hbm.at[t, pl.ds(b0 + h * K, K)], osems[b]).wait()

    def add_pos(c, b):
        buf = bufs[b]
        t = c // HALVES

        def row_body(r, carry):
            for i in range(VECS):
                sl = pl.ds(i * LANES, LANES)
                buf[r, sl] = buf[r, sl] + pos_v[t, sl]
            return carry

        lax.fori_loop(0, K, row_body, 0, unroll=False)

    start_gather(0, 0)
    start_gather(1, 1)

    def chunk_step(c, b):
        nb = (b + 2) % NBUF

        @pl.when(c >= 2)
        def _():
            wait_scatter(c - 2, nb)

        start_gather(c + 2, nb)
        wait_gather(c, b)
        add_pos(c, b)
        start_scatter(c, b)

    def quad_body(i, carry):
        c0 = i * NBUF
        for bb in range(NBUF):
            chunk_step(c0 + bb, bb)
        return carry

    lax.fori_loop(0, MAIN // NBUF, quad_body, 0, unroll=False)

    for c in range(MAIN, NCHT):
        b = c % NBUF
        wait_scatter(c - 2, (b + 2) % NBUF)
        wait_gather(c, b)
        add_pos(c, b)
        start_scatter(c, b)

    for c in range(NCHT - 2, NCHT):
        wait_scatter(c, c % NBUF)


@jax.jit
def _sc_embed(xwt, token_embed, pos_embed):
    mesh = plsc.VectorSubcoreMesh(
        core_axis_name="c", subcore_axis_name="s",
        num_cores=NC, num_subcores=NS)
    fn = pl.kernel(
        _sc_embed_body,
        out_type=jax.ShapeDtypeStruct((CTX, BATCH, EMBED), jnp.float32),
        mesh=mesh,
        scratch_types=[
            pltpu.VMEM((CTX, B_PER_W), jnp.int32),
            pltpu.VMEM((CTX, EMBED), jnp.float32),
            pltpu.VMEM((K, EMBED), jnp.float32),
            pltpu.VMEM((K, EMBED), jnp.float32),
            pltpu.VMEM((K, EMBED), jnp.float32),
            pltpu.VMEM((K, EMBED), jnp.float32),
            pltpu.SemaphoreType.DMA,
            pltpu.SemaphoreType.DMA,
            pltpu.SemaphoreType.DMA,
            pltpu.SemaphoreType.DMA,
            pltpu.SemaphoreType.DMA,
            pltpu.SemaphoreType.DMA,
            pltpu.SemaphoreType.DMA,
            pltpu.SemaphoreType.DMA,
        ],
    )
    return fn(xwt, token_embed, pos_embed)


def kernel(x, token_embed, pos_embed):
    xwt = x.astype(jnp.int32).reshape(NW, B_PER_W, CTX).transpose(0, 2, 1)
    out_t = _sc_embed(xwt, token_embed, pos_embed)
    return out_t.transpose(1, 0, 2)


# speedup vs baseline: 1.6014x; 1.0053x over previous
import jax
import jax.numpy as jnp
from jax import lax
from jax.experimental import pallas as pl
from jax.experimental.pallas import tpu as pltpu
from jax.experimental.pallas import tpu_sc as plsc

VOCAB = 49408
EMBED = 768
CTX = 77
BATCH = 1024

NC = 2
NS = 16
NW = NC * NS
B_PER_W = BATCH // NW

K = 16
HALVES = B_PER_W // K
NCHT = CTX * HALVES
NBUF = 7
LOOK = 2
MAIN = NCHT - NBUF
LANES = 16
VECS = EMBED // LANES


def _sc_embed_body(x_hbm, tok_hbm, pos_hbm, out_hbm,
                   idx_v, p0, p1, p2, p3, p4, p5, p6,
                   buf0, buf1, buf2, buf3, buf4, buf5, buf6,
                   g0, g1, g2, g3, g4, g5, g6, o0, o1, o2, o3, o4, o5, o6):
    posb = (p0, p1, p2, p3, p4, p5, p6)
    bufs = (buf0, buf1, buf2, buf3, buf4, buf5, buf6)
    gsems = (g0, g1, g2, g3, g4, g5, g6)
    osems = (o0, o1, o2, o3, o4, o5, o6)

    wid = lax.axis_index("s") * NC + lax.axis_index("c")
    b0 = wid * B_PER_W

    pltpu.sync_copy(x_hbm.at[wid], idx_v)

    def start_gather(c, b):
        t = c // HALVES
        h = c % HALVES
        pltpu.async_copy(
            tok_hbm.at[idx_v.at[t, pl.ds(h * K, K)]], bufs[b], gsems[b])
        pltpu.async_copy(
            pos_hbm.at[pl.ds(t * EMBED, EMBED)], posb[b], gsems[b])

    def wait_gather(c, b):
        t = c // HALVES
        h = c % HALVES
        pltpu.make_async_copy(
            tok_hbm.at[idx_v.at[t, pl.ds(h * K, K)]], bufs[b],
            gsems[b]).wait()
        pltpu.make_async_copy(
            pos_hbm.at[pl.ds(t * EMBED, EMBED)], posb[b],
            gsems[b]).wait()

    def start_scatter(c, b):
        t = c // HALVES
        h = c % HALVES
        pltpu.async_copy(
            bufs[b], out_hbm.at[t, pl.ds(b0 + h * K, K)], osems[b])

    def wait_scatter(c, b):
        t = c // HALVES
        h = c % HALVES
        pltpu.make_async_copy(
            bufs[b], out_hbm.at[t, pl.ds(b0 + h * K, K)], osems[b]).wait()

    def add_pos(c, b):
        buf = bufs[b]
        prow = posb[b]

        def row_body(r, carry):
            for i in range(VECS):
                sl = pl.ds(i * LANES, LANES)
                buf[r, sl] = buf[r, sl] + prow[sl]
            return carry

        lax.fori_loop(0, K, row_body, 0, unroll=False)

    for c in range(LOOK):
        start_gather(c, c)

    def chunk_step(c, b):
        nb = (b + LOOK) % NBUF

        @pl.when(c >= NBUF - LOOK)
        def _():
            wait_scatter(c + LOOK - NBUF, nb)

        start_gather(c + LOOK, nb)
        wait_gather(c, b)
        add_pos(c, b)
        start_scatter(c, b)

    def ring_body(i, carry):
        c0 = i * NBUF
        for bb in range(NBUF):
            chunk_step(c0 + bb, bb)
        return carry

    lax.fori_loop(0, MAIN // NBUF, ring_body, 0, unroll=False)

    for c in range(MAIN, NCHT):
        b = c % NBUF
        if c + LOOK < NCHT:
            nb = (b + LOOK) % NBUF
            wait_scatter(c + LOOK - NBUF, nb)
            start_gather(c + LOOK, nb)
        wait_gather(c, b)
        add_pos(c, b)
        start_scatter(c, b)

    for c in range(NCHT - (NBUF - LOOK) - LOOK, NCHT):
        wait_scatter(c, c % NBUF)


@jax.jit
def _sc_embed(xwt, token_embed, pos_flat):
    mesh = plsc.VectorSubcoreMesh(
        core_axis_name="c", subcore_axis_name="s",
        num_cores=NC, num_subcores=NS)
    fn = pl.kernel(
        _sc_embed_body,
        out_type=jax.ShapeDtypeStruct((CTX, BATCH, EMBED), jnp.float32),
        mesh=mesh,
        scratch_types=(
            [pltpu.VMEM((CTX, B_PER_W), jnp.int32)]
            + [pltpu.VMEM((EMBED,), jnp.float32) for _ in range(NBUF)]
            + [pltpu.VMEM((K, EMBED), jnp.float32) for _ in range(NBUF)]
            + [pltpu.SemaphoreType.DMA for _ in range(2 * NBUF)]
        ),
    )
    return fn(xwt, token_embed, pos_flat)


def kernel(x, token_embed, pos_embed):
    xwt = x.astype(jnp.int32).reshape(NW, B_PER_W, CTX).transpose(0, 2, 1)
    out_t = _sc_embed(xwt, token_embed, pos_embed.reshape(-1))
    return out_t.transpose(1, 0, 2)


# speedup vs baseline: 4.2219x; 2.6364x over previous
import jax
import jax.numpy as jnp
from jax import lax
from jax.experimental import pallas as pl
from jax.experimental.pallas import tpu as pltpu
from jax.experimental.pallas import tpu_sc as plsc

VOCAB = 49408
EMBED = 768
CTX = 77
BATCH = 1024

NC = 2
NS = 16
NW = NC * NS
B_PER_W = BATCH // NW

K = 16
HALVES = B_PER_W // K
NCHT = CTX * HALVES
NBUF = 7
LOOK = 2
MAIN = NCHT - NBUF
LANES = 16
VECS = EMBED // LANES


def _sc_embed_body(x_hbm, tok_hbm, pos_hbm, out_hbm,
                   idx_v, p0, p1, p2, p3, p4, p5, p6,
                   buf0, buf1, buf2, buf3, buf4, buf5, buf6,
                   g0, g1, g2, g3, g4, g5, g6, o0, o1, o2, o3, o4, o5, o6):
    posb = (p0, p1, p2, p3, p4, p5, p6)
    bufs = (buf0, buf1, buf2, buf3, buf4, buf5, buf6)
    gsems = (g0, g1, g2, g3, g4, g5, g6)
    osems = (o0, o1, o2, o3, o4, o5, o6)

    wid = lax.axis_index("s") * NC + lax.axis_index("c")
    b0 = wid * B_PER_W

    pltpu.sync_copy(x_hbm.at[wid], idx_v)

    def start_gather(c, b):
        t = c // HALVES
        h = c % HALVES
        pltpu.async_copy(
            tok_hbm.at[idx_v.at[t, pl.ds(h * K, K)]], bufs[b], gsems[b])
        pltpu.async_copy(
            pos_hbm.at[pl.ds(t * EMBED, EMBED)], posb[b], gsems[b])

    def wait_gather(c, b):
        t = c // HALVES
        h = c % HALVES
        pltpu.make_async_copy(
            tok_hbm.at[idx_v.at[t, pl.ds(h * K, K)]], bufs[b],
            gsems[b]).wait()
        pltpu.make_async_copy(
            pos_hbm.at[pl.ds(t * EMBED, EMBED)], posb[b],
            gsems[b]).wait()

    def start_scatter(c, b):
        t = c // HALVES
        h = c % HALVES
        pltpu.async_copy(
            bufs[b], out_hbm.at[t, pl.ds(b0 + h * K, K)], osems[b])

    def wait_scatter(c, b):
        t = c // HALVES
        h = c % HALVES
        pltpu.make_async_copy(
            bufs[b], out_hbm.at[t, pl.ds(b0 + h * K, K)], osems[b]).wait()

    def add_pos(c, b):
        buf = bufs[b]
        prow = posb[b]
        pvecs = [prow[pl.ds(i * LANES, LANES)] for i in range(VECS)]

        def row_body(r, carry):
            for i in range(VECS):
                plsc.addupdate(buf.at[r, pl.ds(i * LANES, LANES)], pvecs[i])
            return carry

        lax.fori_loop(0, K, row_body, 0, unroll=2)

    for c in range(LOOK):
        start_gather(c, c)

    def chunk_step(c, b):
        nb = (b + LOOK) % NBUF

        @pl.when(c >= NBUF - LOOK)
        def _():
            wait_scatter(c + LOOK - NBUF, nb)

        start_gather(c + LOOK, nb)
        wait_gather(c, b)
        add_pos(c, b)
        start_scatter(c, b)

    def ring_body(i, carry):
        c0 = i * NBUF
        for bb in range(NBUF):
            chunk_step(c0 + bb, bb)
        return carry

    lax.fori_loop(0, MAIN // NBUF, ring_body, 0, unroll=False)

    for c in range(MAIN, NCHT):
        b = c % NBUF
        if c + LOOK < NCHT:
            nb = (b + LOOK) % NBUF
            wait_scatter(c + LOOK - NBUF, nb)
            start_gather(c + LOOK, nb)
        wait_gather(c, b)
        add_pos(c, b)
        start_scatter(c, b)

    for c in range(NCHT - (NBUF - LOOK) - LOOK, NCHT):
        wait_scatter(c, c % NBUF)


@jax.jit
def _sc_embed(xwt, token_embed, pos_flat):
    mesh = plsc.VectorSubcoreMesh(
        core_axis_name="c", subcore_axis_name="s",
        num_cores=NC, num_subcores=NS)
    fn = pl.kernel(
        _sc_embed_body,
        out_type=jax.ShapeDtypeStruct((CTX, BATCH, EMBED), jnp.float32),
        mesh=mesh,
        scratch_types=(
            [pltpu.VMEM((CTX, B_PER_W), jnp.int32)]
            + [pltpu.VMEM((EMBED,), jnp.float32) for _ in range(NBUF)]
            + [pltpu.VMEM((K, EMBED), jnp.float32) for _ in range(NBUF)]
            + [pltpu.SemaphoreType.DMA for _ in range(2 * NBUF)]
        ),
    )
    return fn(xwt, token_embed, pos_flat)


def kernel(x, token_embed, pos_embed):
    xwt = x.astype(jnp.int32).reshape(NW, B_PER_W, CTX).transpose(0, 2, 1)
    out_t = _sc_embed(xwt, token_embed, pos_embed.reshape(-1))
    return out_t.transpose(1, 0, 2)


# speedup vs baseline: 4.8229x; 1.1424x over previous
import jax
import jax.numpy as jnp
from jax import lax
from jax.experimental import pallas as pl
from jax.experimental.pallas import tpu as pltpu
from jax.experimental.pallas import tpu_sc as plsc

VOCAB = 49408
EMBED = 768
CTX = 77
BATCH = 1024

NC = 2
NS = 16
NW = NC * NS
B_PER_W = BATCH // NW

K = 32
HALVES = B_PER_W // K
NCHT = CTX * HALVES
NBUF = 4
LOOK = 2
MAIN = ((NCHT - NBUF) // NBUF) * NBUF
LANES = 16
VECS = EMBED // LANES


def _sc_embed_body(x_hbm, tok_hbm, pos_hbm, out_hbm,
                   idx_v, p0, p1, p2, p3,
                   buf0, buf1, buf2, buf3,
                   g0, g1, g2, g3, o0, o1, o2, o3):
    posb = (p0, p1, p2, p3)
    bufs = (buf0, buf1, buf2, buf3)
    gsems = (g0, g1, g2, g3)
    osems = (o0, o1, o2, o3)

    wid = lax.axis_index("s") * NC + lax.axis_index("c")
    b0 = wid * B_PER_W

    pltpu.sync_copy(x_hbm.at[wid], idx_v)

    def start_gather(c, b):
        t = c // HALVES
        h = c % HALVES
        pltpu.async_copy(
            tok_hbm.at[idx_v.at[t, pl.ds(h * K, K)]], bufs[b], gsems[b])
        pltpu.async_copy(
            pos_hbm.at[pl.ds(t * EMBED, EMBED)], posb[b], gsems[b])

    def wait_gather(c, b):
        t = c // HALVES
        h = c % HALVES
        pltpu.make_async_copy(
            tok_hbm.at[idx_v.at[t, pl.ds(h * K, K)]], bufs[b],
            gsems[b]).wait()
        pltpu.make_async_copy(
            pos_hbm.at[pl.ds(t * EMBED, EMBED)], posb[b],
            gsems[b]).wait()

    def start_scatter(c, b):
        t = c // HALVES
        h = c % HALVES
        pltpu.async_copy(
            bufs[b], out_hbm.at[t, pl.ds(b0 + h * K, K)], osems[b])

    def wait_scatter(c, b):
        t = c // HALVES
        h = c % HALVES
        pltpu.make_async_copy(
            bufs[b], out_hbm.at[t, pl.ds(b0 + h * K, K)], osems[b]).wait()

    def add_pos(c, b):
        buf = bufs[b]
        prow = posb[b]
        pvecs = [prow[pl.ds(i * LANES, LANES)] for i in range(VECS)]

        def row_body(r, carry):
            for i in range(VECS):
                plsc.addupdate(buf.at[r, pl.ds(i * LANES, LANES)], pvecs[i])
            return carry

        lax.fori_loop(0, K, row_body, 0, unroll=2)

    for c in range(LOOK):
        start_gather(c, c)

    def chunk_step(c, b):
        nb = (b + LOOK) % NBUF

        @pl.when(c >= NBUF - LOOK)
        def _():
            wait_scatter(c + LOOK - NBUF, nb)

        start_gather(c + LOOK, nb)
        wait_gather(c, b)
        add_pos(c, b)
        start_scatter(c, b)

    def ring_body(i, carry):
        c0 = i * NBUF
        for bb in range(NBUF):
            chunk_step(c0 + bb, bb)
        return carry

    lax.fori_loop(0, MAIN // NBUF, ring_body, 0, unroll=False)

    for c in range(MAIN, NCHT):
        b = c % NBUF
        if c + LOOK < NCHT:
            nb = (b + LOOK) % NBUF
            wait_scatter(c + LOOK - NBUF, nb)
            start_gather(c + LOOK, nb)
        wait_gather(c, b)
        add_pos(c, b)
        start_scatter(c, b)

    for c in range(NCHT - (NBUF - LOOK) - LOOK, NCHT):
        wait_scatter(c, c % NBUF)


@jax.jit
def _sc_embed(xwt, token_embed, pos_flat):
    mesh = plsc.VectorSubcoreMesh(
        core_axis_name="c", subcore_axis_name="s",
        num_cores=NC, num_subcores=NS)
    fn = pl.kernel(
        _sc_embed_body,
        out_type=jax.ShapeDtypeStruct((CTX, BATCH, EMBED), jnp.float32),
        mesh=mesh,
        scratch_types=(
            [pltpu.VMEM((CTX, B_PER_W), jnp.int32)]
            + [pltpu.VMEM((EMBED,), jnp.float32) for _ in range(NBUF)]
            + [pltpu.VMEM((K, EMBED), jnp.float32) for _ in range(NBUF)]
            + [pltpu.SemaphoreType.DMA for _ in range(2 * NBUF)]
        ),
    )
    return fn(xwt, token_embed, pos_flat)


def kernel(x, token_embed, pos_embed):
    xwt = x.astype(jnp.int32).reshape(NW, B_PER_W, CTX).transpose(0, 2, 1)
    out_t = _sc_embed(xwt, token_embed, pos_embed.reshape(-1))
    return out_t.transpose(1, 0, 2)
